# baseline (speedup 1.0000x reference)
import functools

import jax
import jax.numpy as jnp
from jax import lax
from jax.experimental import pallas as pl
from jax.experimental.pallas import tpu as pltpu
from jax.experimental.pallas import tpu_sc as plsc

N = 10000
E = 320000
G = 64
D = 128

NC = 2
NS = 16
NT = NC * NS
EPT = E // NT
CH = 80
NCH = EPT // CH
WS = 640
STRIDE = 624
ZR = 128
DEGW = 16

RB = 1000

@functools.cache
def _mesh():
    return plsc.VectorSubcoreMesh(
        core_axis_name="c", subcore_axis_name="s", num_cores=NC, num_subcores=NS
    )


@functools.cache
def _sc_degree_kernel():
    return pl.kernel(
        _sc_degree_body,
        out_type=jax.ShapeDtypeStruct((NC, N, DEGW), jnp.float32),
        mesh=_mesh(),
        scratch_types=[
            pltpu.VMEM((NCH, CH), jnp.int32),
            pltpu.VMEM((CH, DEGW), jnp.float32),
            pltpu.VMEM((ZR, DEGW), jnp.float32),
            pltpu.VMEM_SHARED((N, DEGW), jnp.float32),
        ],
    )


def _sc_degree(dst):
    return _sc_degree_kernel()(dst)


def _sc_degree_body(dst_hbm, out_hbm, dst_v, ones_v, zer_v, acc_sh):
    c = lax.axis_index("c")
    s = lax.axis_index("s")
    t = c * NS + s
    one16 = jnp.full((16,), 1.0, jnp.float32)
    zero16 = jnp.zeros((16,), jnp.float32)

    def fill(i, _):
        ones_v[i, :] = one16
        return 0

    lax.fori_loop(0, CH, fill, 0)

    def zfill(i, _):
        zer_v[i, :] = zero16
        return 0

    lax.fori_loop(0, ZR, zfill, 0)

    row0 = s * STRIDE
    for k in range(WS // ZR):
        pltpu.sync_copy(zer_v, acc_sh.at[pl.ds(row0 + k * ZR, ZR)])
    pltpu.sync_copy(dst_hbm.at[t], dst_v)
    plsc.subcore_barrier()

    def body(k, _):
        pltpu.sync_copy(ones_v, acc_sh.at[dst_v.at[k]], add=True)
        return 0

    lax.fori_loop(0, NCH, body, 0)
    plsc.subcore_barrier()
    pltpu.sync_copy(acc_sh.at[pl.ds(row0, WS)], out_hbm.at[c, pl.ds(row0, WS)])


@functools.cache
def _sc_propagate_kernel():
    return pl.kernel(
        _sc_propagate_body,
        out_type=jax.ShapeDtypeStruct((NC, N, D), jnp.float32),
        mesh=_mesh(),
        scratch_types=[
            pltpu.VMEM((EPT,), jnp.int32),
            pltpu.VMEM((NCH, CH), jnp.int32),
            pltpu.VMEM((CH, D), jnp.float32),
            pltpu.VMEM((CH, D), jnp.float32),
            pltpu.VMEM_SHARED((N, D), jnp.float32),
            pltpu.SemaphoreType.DMA,
            pltpu.SemaphoreType.DMA,
            pltpu.SemaphoreType.DMA,
            pltpu.SemaphoreType.DMA,
        ],
    )


def _sc_propagate(g, src, dst):
    return _sc_propagate_kernel()(g, src, dst)


def _sc_propagate_body(g_hbm, src_hbm, dst_hbm, out_hbm, src_v, dst_v, rows_a, rows_b, acc_sh, ga, gb, sa, sb):
    c = lax.axis_index("c")
    s = lax.axis_index("s")
    t = c * NS + s
    zero16 = jnp.zeros((16,), jnp.float32)

    def zfill(i, _):
        for j in range(D // 16):
            rows_a[i, pl.ds(j * 16, 16)] = zero16
        return 0

    lax.fori_loop(0, CH, zfill, 0)
    row0 = s * STRIDE
    for k in range(WS // CH):
        pltpu.async_copy(rows_a, acc_sh.at[pl.ds(row0 + k * CH, CH)], sa)
    pltpu.sync_copy(src_hbm.at[pl.ds(t * EPT, EPT)], src_v)
    pltpu.sync_copy(dst_hbm.at[t], dst_v)
    for k in range(WS // CH):
        pltpu.make_async_copy(rows_a, acc_sh.at[pl.ds(row0 + k * CH, CH)], sa).wait()
    plsc.subcore_barrier()

    def gissue(k, buf, sem):
        off = pl.multiple_of(k * CH, 8)
        pltpu.async_copy(g_hbm.at[src_v.at[pl.ds(off, CH)]], buf, sem)

    def gwait(buf, sem):
        pltpu.make_async_copy(g_hbm.at[pl.ds(0, CH)], buf, sem).wait()

    def sissue(k, buf, sem):
        pltpu.async_copy(buf, acc_sh.at[dst_v.at[k]], sem, add=True)

    def swait(k, buf, sem):
        pltpu.make_async_copy(buf, acc_sh.at[dst_v.at[k]], sem).wait()

    gissue(0, rows_a, ga)

    def body(i, _):
        k0 = 2 * i

        @pl.when(i > 0)
        def _():
            swait(k0 - 1, rows_b, sb)

        gissue(k0 + 1, rows_b, gb)
        gwait(rows_a, ga)
        sissue(k0, rows_a, sa)
        gwait(rows_b, gb)
        sissue(k0 + 1, rows_b, sb)
        swait(k0, rows_a, sa)
        gissue(k0 + 2, rows_a, ga)
        return 0

    if NCH % 2:
        lax.fori_loop(0, (NCH - 1) // 2, body, 0)
        swait(NCH - 2, rows_b, sb)
        gwait(rows_a, ga)
        sissue(NCH - 1, rows_a, sa)
        swait(NCH - 1, rows_a, sa)
    else:
        lax.fori_loop(0, NCH // 2 - 1, body, 0)
        swait(NCH - 3, rows_b, sb)
        gissue(NCH - 1, rows_b, gb)
        gwait(rows_a, ga)
        sissue(NCH - 2, rows_a, sa)
        swait(NCH - 2, rows_a, sa)
        gwait(rows_b, gb)
        sissue(NCH - 1, rows_b, sb)
        swait(NCH - 1, rows_b, sb)
    plsc.subcore_barrier()
    pltpu.sync_copy(acc_sh.at[pl.ds(row0, WS)], out_hbm.at[c, pl.ds(row0, WS)])


def _dinv_block(d_ref):
    return lax.rsqrt(d_ref[0, :, 0:1] + d_ref[1, :, 0:1] + 1.0)


def _encode_body(x_ref, d_ref, w_ref, b_ref, o_ref):
    h = jnp.maximum(
        jnp.dot(x_ref[...], w_ref[...], preferred_element_type=jnp.float32) + b_ref[...],
        0.0,
    )
    o_ref[...] = _dinv_block(d_ref) * h


def _tc_encode(x, deg2, W, b):
    return pl.pallas_call(
        _encode_body,
        grid=(N // RB,),
        in_specs=[
            pl.BlockSpec((RB, D), lambda i: (i, 0)),
            pl.BlockSpec((NC, RB, DEGW), lambda i: (0, i, 0)),
            pl.BlockSpec((D, D), lambda i: (0, 0)),
            pl.BlockSpec((1, D), lambda i: (0, 0)),
        ],
        out_specs=pl.BlockSpec((RB, D), lambda i: (i, 0)),
        out_shape=jax.ShapeDtypeStruct((N, D), jnp.float32),
    )(x, deg2, W, b)


def _conv_body(s_ref, g_ref, d_ref, w_ref, b_ref, o_ref):
    dinv = _dinv_block(d_ref)
    z = dinv * (s_ref[0] + s_ref[1] + g_ref[...])
    h = jnp.maximum(
        jnp.dot(z, w_ref[...], preferred_element_type=jnp.float32) + b_ref[...],
        0.0,
    )
    o_ref[...] = dinv * h


def _tc_conv(s, g, deg2, W, b):
    return pl.pallas_call(
        _conv_body,
        grid=(N // RB,),
        in_specs=[
            pl.BlockSpec((NC, RB, D), lambda i: (0, i, 0)),
            pl.BlockSpec((RB, D), lambda i: (i, 0)),
            pl.BlockSpec((NC, RB, DEGW), lambda i: (0, i, 0)),
            pl.BlockSpec((D, D), lambda i: (0, 0)),
            pl.BlockSpec((1, D), lambda i: (0, 0)),
        ],
        out_specs=pl.BlockSpec((RB, D), lambda i: (i, 0)),
        out_shape=jax.ShapeDtypeStruct((N, D), jnp.float32),
    )(s, g, deg2, W, b)


def _final_body(s_ref, g_ref, d_ref, bat_ref, w3_ref, b3_ref, wd_ref, bd_ref, o_ref, acc_ref):
    i = pl.program_id(0)
    dinv = _dinv_block(d_ref)
    z = dinv * (s_ref[0] + s_ref[1] + g_ref[...])
    h3 = jnp.maximum(
        jnp.dot(z, w3_ref[...], preferred_element_type=jnp.float32) + b3_ref[...],
        0.0,
    )
    bat = bat_ref[0, 0, :]
    ind = (lax.broadcasted_iota(jnp.int32, (G, RB), 0) == bat[None, :]).astype(jnp.float32)
    part = jnp.dot(ind, h3, preferred_element_type=jnp.float32)

    @pl.when(i == 0)
    def _():
        acc_ref[...] = jnp.zeros_like(acc_ref)

    acc_ref[...] += part

    @pl.when(i == pl.num_programs(0) - 1)
    def _():
        o_ref[...] = (
            jnp.dot(acc_ref[...], wd_ref[...], preferred_element_type=jnp.float32)
            + bd_ref[...]
        )


def _tc_final(s, g, deg2, bat3, W3, b3, Wd, bd):
    return pl.pallas_call(
        _final_body,
        grid=(N // RB,),
        in_specs=[
            pl.BlockSpec((NC, RB, D), lambda i: (0, i, 0)),
            pl.BlockSpec((RB, D), lambda i: (i, 0)),
            pl.BlockSpec((NC, RB, DEGW), lambda i: (0, i, 0)),
            pl.BlockSpec((1, 1, RB), lambda i: (i, 0, 0)),
            pl.BlockSpec((D, D), lambda i: (0, 0)),
            pl.BlockSpec((1, D), lambda i: (0, 0)),
            pl.BlockSpec((D, D), lambda i: (0, 0)),
            pl.BlockSpec((1, D), lambda i: (0, 0)),
        ],
        out_specs=pl.BlockSpec((G, D), lambda i: (0, 0)),
        out_shape=jax.ShapeDtypeStruct((G, D), jnp.float32),
        scratch_shapes=[pltpu.VMEM((G, D), jnp.float32)],
    )(s, g, deg2, bat3, W3, b3, Wd, bd)


def kernel(x, edge_index, batch, W_enc, b_enc, W_c1, b_c1, W_c2, b_c2, W_c3, b_c3, W_dec, b_dec):
    src = edge_index[0]
    dst = edge_index[1].reshape(NT, NCH, CH)
    deg2 = _sc_degree(dst)
    g0 = _tc_encode(x, deg2, W_enc, b_enc.reshape(1, D))
    s = _sc_propagate(g0, src, dst)
    g1 = _tc_conv(s, g0, deg2, W_c1, b_c1.reshape(1, D))
    s = _sc_propagate(g1, src, dst)
    g2 = _tc_conv(s, g1, deg2, W_c2, b_c2.reshape(1, D))
    s = _sc_propagate(g2, src, dst)
    bat3 = batch.reshape(N // RB, 1, RB)
    return _tc_final(s, g2, deg2, bat3, W_c3, b_c3.reshape(1, D), W_dec, b_dec.reshape(1, D))


# speedup vs baseline: 1.2361x; 1.2361x over previous
import functools

import jax
import jax.numpy as jnp
from jax import lax
from jax.experimental import pallas as pl
from jax.experimental.pallas import tpu as pltpu
from jax.experimental.pallas import tpu_sc as plsc

N = 10000
E = 320000
G = 64
D = 128

NC = 2
NS = 16
NT = NC * NS
EPT = E // NT
CH = 80
NCH = EPT // CH
WS = 640
STRIDE = 624
ZR = 128
DEGW = 16

RB = 1000

@functools.cache
def _mesh():
    return plsc.VectorSubcoreMesh(
        core_axis_name="c", subcore_axis_name="s", num_cores=NC, num_subcores=NS
    )


@functools.cache
def _sc_degree_kernel():
    return pl.kernel(
        _sc_degree_body,
        out_type=jax.ShapeDtypeStruct((NC, N, DEGW), jnp.float32),
        mesh=_mesh(),
        scratch_types=[
            pltpu.VMEM((NCH, CH), jnp.int32),
            pltpu.VMEM((CH, DEGW), jnp.float32),
            pltpu.VMEM((ZR, DEGW), jnp.float32),
            pltpu.VMEM_SHARED((N, DEGW), jnp.float32),
        ],
    )


def _sc_degree(dst):
    return _sc_degree_kernel()(dst)


def _sc_degree_body(dst_hbm, out_hbm, dst_v, ones_v, zer_v, acc_sh):
    c = lax.axis_index("c")
    s = lax.axis_index("s")
    t = c * NS + s
    one16 = jnp.full((16,), 1.0, jnp.float32)
    zero16 = jnp.zeros((16,), jnp.float32)

    def fill(i, _):
        ones_v[i, :] = one16
        return 0

    lax.fori_loop(0, CH, fill, 0)

    def zfill(i, _):
        zer_v[i, :] = zero16
        return 0

    lax.fori_loop(0, ZR, zfill, 0)

    row0 = s * STRIDE
    for k in range(WS // ZR):
        pltpu.sync_copy(zer_v, acc_sh.at[pl.ds(row0 + k * ZR, ZR)])
    pltpu.sync_copy(dst_hbm.at[t], dst_v)
    plsc.subcore_barrier()

    def body(k, _):
        pltpu.sync_copy(ones_v, acc_sh.at[dst_v.at[k]], add=True)
        return 0

    lax.fori_loop(0, NCH, body, 0)
    plsc.subcore_barrier()
    pltpu.sync_copy(acc_sh.at[pl.ds(row0, WS)], out_hbm.at[c, pl.ds(row0, WS)])


@functools.cache
def _sc_propagate_kernel():
    return pl.kernel(
        _sc_propagate_body,
        out_type=jax.ShapeDtypeStruct((NC, N, D), jnp.float32),
        mesh=_mesh(),
        scratch_types=[
            pltpu.VMEM((EPT,), jnp.int32),
            pltpu.VMEM((NCH, CH), jnp.int32),
            pltpu.VMEM((CH, D), jnp.float32),
            pltpu.VMEM((CH, D), jnp.float32),
            pltpu.VMEM_SHARED((N, D), jnp.float32),
            pltpu.SemaphoreType.DMA,
            pltpu.SemaphoreType.DMA,
            pltpu.SemaphoreType.DMA,
            pltpu.SemaphoreType.DMA,
        ],
    )


def _sc_propagate(g, src, dst):
    return _sc_propagate_kernel()(g, src, dst)


def _sc_propagate_body(g_hbm, src_hbm, dst_hbm, out_hbm, src_v, dst_v, rows_a, rows_b, acc_sh, ga, gb, sa, sb):
    c = lax.axis_index("c")
    s = lax.axis_index("s")
    t = c * NS + s
    zero16 = jnp.zeros((16,), jnp.float32)

    def zfill(i, _):
        for j in range(D // 16):
            rows_a[i, pl.ds(j * 16, 16)] = zero16
        return 0

    lax.fori_loop(0, CH, zfill, 0)
    row0 = s * STRIDE
    for k in range(WS // CH):
        pltpu.async_copy(rows_a, acc_sh.at[pl.ds(row0 + k * CH, CH)], sa)
    pltpu.sync_copy(src_hbm.at[pl.ds(t * EPT, EPT)], src_v)
    pltpu.sync_copy(dst_hbm.at[t], dst_v)
    for k in range(WS // CH):
        pltpu.make_async_copy(rows_a, acc_sh.at[pl.ds(row0 + k * CH, CH)], sa).wait()
    plsc.subcore_barrier()

    def gissue(k, buf, sem):
        off = pl.multiple_of(k * CH, 8)
        pltpu.async_copy(g_hbm.at[src_v.at[pl.ds(off, CH)]], buf, sem)

    def gwait(buf, sem):
        pltpu.make_async_copy(g_hbm.at[pl.ds(0, CH)], buf, sem).wait()

    def sissue(k, buf, sem):
        pltpu.async_copy(buf, acc_sh.at[dst_v.at[k]], sem, add=True)

    def swait(k, buf, sem):
        pltpu.make_async_copy(buf, acc_sh.at[dst_v.at[k]], sem).wait()

    gissue(0, rows_a, ga)

    def body(i, _):
        k0 = 2 * i

        @pl.when(i > 0)
        def _():
            swait(k0 - 1, rows_b, sb)

        gissue(k0 + 1, rows_b, gb)
        gwait(rows_a, ga)
        sissue(k0, rows_a, sa)
        swait(k0, rows_a, sa)
        gissue(k0 + 2, rows_a, ga)
        gwait(rows_b, gb)
        sissue(k0 + 1, rows_b, sb)
        return 0

    if NCH % 2:
        lax.fori_loop(0, (NCH - 1) // 2, body, 0)
        swait(NCH - 2, rows_b, sb)
        gwait(rows_a, ga)
        sissue(NCH - 1, rows_a, sa)
        swait(NCH - 1, rows_a, sa)
    else:
        lax.fori_loop(0, NCH // 2 - 1, body, 0)
        swait(NCH - 3, rows_b, sb)
        gissue(NCH - 1, rows_b, gb)
        gwait(rows_a, ga)
        sissue(NCH - 2, rows_a, sa)
        swait(NCH - 2, rows_a, sa)
        gwait(rows_b, gb)
        sissue(NCH - 1, rows_b, sb)
        swait(NCH - 1, rows_b, sb)
    plsc.subcore_barrier()
    pltpu.sync_copy(acc_sh.at[pl.ds(row0, WS)], out_hbm.at[c, pl.ds(row0, WS)])


def _dinv_block(d_ref):
    return lax.rsqrt(d_ref[0, :, 0:1] + d_ref[1, :, 0:1] + 1.0)


def _encode_body(x_ref, d_ref, w_ref, b_ref, o_ref):
    h = jnp.maximum(
        jnp.dot(x_ref[...], w_ref[...], preferred_element_type=jnp.float32) + b_ref[...],
        0.0,
    )
    o_ref[...] = _dinv_block(d_ref) * h


def _tc_encode(x, deg2, W, b):
    return pl.pallas_call(
        _encode_body,
        grid=(N // RB,),
        in_specs=[
            pl.BlockSpec((RB, D), lambda i: (i, 0)),
            pl.BlockSpec((NC, RB, DEGW), lambda i: (0, i, 0)),
            pl.BlockSpec((D, D), lambda i: (0, 0)),
            pl.BlockSpec((1, D), lambda i: (0, 0)),
        ],
        out_specs=pl.BlockSpec((RB, D), lambda i: (i, 0)),
        out_shape=jax.ShapeDtypeStruct((N, D), jnp.float32),
    )(x, deg2, W, b)


def _conv_body(s_ref, g_ref, d_ref, w_ref, b_ref, o_ref):
    dinv = _dinv_block(d_ref)
    z = dinv * (s_ref[0] + s_ref[1] + g_ref[...])
    h = jnp.maximum(
        jnp.dot(z, w_ref[...], preferred_element_type=jnp.float32) + b_ref[...],
        0.0,
    )
    o_ref[...] = dinv * h


def _tc_conv(s, g, deg2, W, b):
    return pl.pallas_call(
        _conv_body,
        grid=(N // RB,),
        in_specs=[
            pl.BlockSpec((NC, RB, D), lambda i: (0, i, 0)),
            pl.BlockSpec((RB, D), lambda i: (i, 0)),
            pl.BlockSpec((NC, RB, DEGW), lambda i: (0, i, 0)),
            pl.BlockSpec((D, D), lambda i: (0, 0)),
            pl.BlockSpec((1, D), lambda i: (0, 0)),
        ],
        out_specs=pl.BlockSpec((RB, D), lambda i: (i, 0)),
        out_shape=jax.ShapeDtypeStruct((N, D), jnp.float32),
    )(s, g, deg2, W, b)


def _final_body(s_ref, g_ref, d_ref, bat_ref, w3_ref, b3_ref, wd_ref, bd_ref, o_ref, acc_ref):
    i = pl.program_id(0)
    dinv = _dinv_block(d_ref)
    z = dinv * (s_ref[0] + s_ref[1] + g_ref[...])
    h3 = jnp.maximum(
        jnp.dot(z, w3_ref[...], preferred_element_type=jnp.float32) + b3_ref[...],
        0.0,
    )
    bat = bat_ref[0, 0, :]
    ind = (lax.broadcasted_iota(jnp.int32, (G, RB), 0) == bat[None, :]).astype(jnp.float32)
    part = jnp.dot(ind, h3, preferred_element_type=jnp.float32)

    @pl.when(i == 0)
    def _():
        acc_ref[...] = jnp.zeros_like(acc_ref)

    acc_ref[...] += part

    @pl.when(i == pl.num_programs(0) - 1)
    def _():
        o_ref[...] = (
            jnp.dot(acc_ref[...], wd_ref[...], preferred_element_type=jnp.float32)
            + bd_ref[...]
        )


def _tc_final(s, g, deg2, bat3, W3, b3, Wd, bd):
    return pl.pallas_call(
        _final_body,
        grid=(N // RB,),
        in_specs=[
            pl.BlockSpec((NC, RB, D), lambda i: (0, i, 0)),
            pl.BlockSpec((RB, D), lambda i: (i, 0)),
            pl.BlockSpec((NC, RB, DEGW), lambda i: (0, i, 0)),
            pl.BlockSpec((1, 1, RB), lambda i: (i, 0, 0)),
            pl.BlockSpec((D, D), lambda i: (0, 0)),
            pl.BlockSpec((1, D), lambda i: (0, 0)),
            pl.BlockSpec((D, D), lambda i: (0, 0)),
            pl.BlockSpec((1, D), lambda i: (0, 0)),
        ],
        out_specs=pl.BlockSpec((G, D), lambda i: (0, 0)),
        out_shape=jax.ShapeDtypeStruct((G, D), jnp.float32),
        scratch_shapes=[pltpu.VMEM((G, D), jnp.float32)],
    )(s, g, deg2, bat3, W3, b3, Wd, bd)


def kernel(x, edge_index, batch, W_enc, b_enc, W_c1, b_c1, W_c2, b_c2, W_c3, b_c3, W_dec, b_dec):
    src = edge_index[0]
    dst = edge_index[1].reshape(NT, NCH, CH)
    deg2 = _sc_degree(dst)
    g0 = _tc_encode(x, deg2, W_enc, b_enc.reshape(1, D))
    s = _sc_propagate(g0, src, dst)
    g1 = _tc_conv(s, g0, deg2, W_c1, b_c1.reshape(1, D))
    s = _sc_propagate(g1, src, dst)
    g2 = _tc_conv(s, g1, deg2, W_c2, b_c2.reshape(1, D))
    s = _sc_propagate(g2, src, dst)
    bat3 = batch.reshape(N // RB, 1, RB)
    return _tc_final(s, g2, deg2, bat3, W_c3, b_c3.reshape(1, D), W_dec, b_dec.reshape(1, D))


# speedup vs baseline: 1.2586x; 1.0182x over previous
import functools

import jax
import jax.numpy as jnp
from jax import lax
from jax.experimental import pallas as pl
from jax.experimental.pallas import tpu as pltpu
from jax.experimental.pallas import tpu_sc as plsc

N = 10000
E = 320000
G = 64
D = 128

NC = 2
NS = 16
NT = NC * NS
EPT = E // NT
CH = 80
NCH = EPT // CH
WS = 640
STRIDE = 624
ZR = 128
DEGW = 16

RB = 1000

NBUF = 4
GCH = 25
NG = NCH // GCH

@functools.cache
def _mesh():
    return plsc.VectorSubcoreMesh(
        core_axis_name="c", subcore_axis_name="s", num_cores=NC, num_subcores=NS
    )


@functools.cache
def _sc_degree_kernel():
    return pl.kernel(
        _sc_degree_body,
        out_type=jax.ShapeDtypeStruct((NC, N, DEGW), jnp.float32),
        mesh=_mesh(),
        scratch_types=[
            pltpu.VMEM((NCH, CH), jnp.int32),
            pltpu.VMEM((CH, DEGW), jnp.float32),
            pltpu.VMEM((ZR, DEGW), jnp.float32),
            pltpu.VMEM_SHARED((N, DEGW), jnp.float32),
        ],
    )


def _sc_degree(dst):
    return _sc_degree_kernel()(dst)


def _sc_degree_body(dst_hbm, out_hbm, dst_v, ones_v, zer_v, acc_sh):
    c = lax.axis_index("c")
    s = lax.axis_index("s")
    t = c * NS + s
    one16 = jnp.full((16,), 1.0, jnp.float32)
    zero16 = jnp.zeros((16,), jnp.float32)

    def fill(i, _):
        ones_v[i, :] = one16
        return 0

    lax.fori_loop(0, CH, fill, 0)

    def zfill(i, _):
        zer_v[i, :] = zero16
        return 0

    lax.fori_loop(0, ZR, zfill, 0)

    row0 = s * STRIDE
    for k in range(WS // ZR):
        pltpu.sync_copy(zer_v, acc_sh.at[pl.ds(row0 + k * ZR, ZR)])
    pltpu.sync_copy(dst_hbm.at[t], dst_v)
    plsc.subcore_barrier()

    def body(k, _):
        pltpu.sync_copy(ones_v, acc_sh.at[dst_v.at[k]], add=True)
        return 0

    lax.fori_loop(0, NCH, body, 0)
    plsc.subcore_barrier()
    pltpu.sync_copy(acc_sh.at[pl.ds(row0, WS)], out_hbm.at[c, pl.ds(row0, WS)])


@functools.cache
def _sc_propagate_kernel():
    return pl.kernel(
        _sc_propagate_body,
        out_type=jax.ShapeDtypeStruct((NC, N, D), jnp.float32),
        mesh=_mesh(),
        scratch_types=[
            pltpu.VMEM((GCH * CH,), jnp.int32),
            pltpu.VMEM((GCH, CH), jnp.int32),
            pltpu.VMEM((CH, D), jnp.float32),
            pltpu.VMEM((CH, D), jnp.float32),
            pltpu.VMEM((CH, D), jnp.float32),
            pltpu.VMEM((CH, D), jnp.float32),
            pltpu.VMEM_SHARED((N, D), jnp.float32),
        ]
        + [pltpu.SemaphoreType.DMA] * (2 * NBUF),
    )


def _sc_propagate(g, src, dst):
    return _sc_propagate_kernel()(g, src, dst)


def _sc_propagate_body(g_hbm, src_hbm, dst_hbm, out_hbm, src_v, dst_v, r0, r1, r2, r3, acc_sh, *sems):
    rows = [r0, r1, r2, r3]
    gs = list(sems[:NBUF])
    ss = list(sems[NBUF:])
    c = lax.axis_index("c")
    s = lax.axis_index("s")
    t = c * NS + s
    zero16 = jnp.zeros((16,), jnp.float32)

    def zfill(i, _):
        for j in range(D // 16):
            r0[i, pl.ds(j * 16, 16)] = zero16
        return 0

    lax.fori_loop(0, CH, zfill, 0)
    row0 = s * STRIDE
    for k in range(WS // CH):
        pltpu.async_copy(r0, acc_sh.at[pl.ds(row0 + k * CH, CH)], ss[0])
    for k in range(WS // CH):
        pltpu.make_async_copy(r0, acc_sh.at[pl.ds(row0 + k * CH, CH)], ss[0]).wait()
    plsc.subcore_barrier()

    def gissue(k, buf, sem):
        off = pl.multiple_of(k * CH, 8)
        pltpu.async_copy(g_hbm.at[src_v.at[pl.ds(off, CH)]], buf, sem)

    def gwait(buf, sem):
        pltpu.make_async_copy(g_hbm.at[pl.ds(0, CH)], buf, sem).wait()

    def sissue(k, buf, sem):
        pltpu.async_copy(buf, acc_sh.at[dst_v.at[k]], sem, add=True)

    def swait(k, buf, sem):
        pltpu.make_async_copy(buf, acc_sh.at[dst_v.at[k]], sem).wait()

    for grp in range(NG):
        pltpu.sync_copy(src_hbm.at[pl.ds(t * EPT + grp * GCH * CH, GCH * CH)], src_v)
        pltpu.sync_copy(dst_hbm.at[t, grp], dst_v)
        for b in range(3):
            gissue(b, rows[b], gs[b])

        def quad(i, _):
            for b in range(4):
                k = 4 * i + b
                gwait(rows[b], gs[b])
                sissue(k, rows[b], ss[b])
                bp = (b - 1) % 4

                @pl.when(k >= 1)
                def _():
                    swait(k - 1, rows[bp], ss[bp])

                bn = (b + 3) % 4

                @pl.when(k + 3 <= GCH - 1)
                def _():
                    gissue(k + 3, rows[bn], gs[bn])

            return 0

        lax.fori_loop(0, (GCH - 1) // 4, quad, 0)
        klast = GCH - 1
        gwait(rows[klast % 4], gs[klast % 4])
        sissue(klast, rows[klast % 4], ss[klast % 4])
        swait(klast - 1, rows[(klast - 1) % 4], ss[(klast - 1) % 4])
        swait(klast, rows[klast % 4], ss[klast % 4])

    plsc.subcore_barrier()
    pltpu.sync_copy(acc_sh.at[pl.ds(row0, WS)], out_hbm.at[c, pl.ds(row0, WS)])


def _dinv_block(d_ref):
    return lax.rsqrt(d_ref[0, :, 0:1] + d_ref[1, :, 0:1] + 1.0)


def _encode_body(x_ref, d_ref, w_ref, b_ref, o_ref):
    h = jnp.maximum(
        jnp.dot(x_ref[...], w_ref[...], preferred_element_type=jnp.float32) + b_ref[...],
        0.0,
    )
    o_ref[...] = _dinv_block(d_ref) * h


def _tc_encode(x, deg2, W, b):
    return pl.pallas_call(
        _encode_body,
        grid=(N // RB,),
        in_specs=[
            pl.BlockSpec((RB, D), lambda i: (i, 0)),
            pl.BlockSpec((NC, RB, DEGW), lambda i: (0, i, 0)),
            pl.BlockSpec((D, D), lambda i: (0, 0)),
            pl.BlockSpec((1, D), lambda i: (0, 0)),
        ],
        out_specs=pl.BlockSpec((RB, D), lambda i: (i, 0)),
        out_shape=jax.ShapeDtypeStruct((N, D), jnp.float32),
    )(x, deg2, W, b)


def _conv_body(s_ref, g_ref, d_ref, w_ref, b_ref, o_ref):
    dinv = _dinv_block(d_ref)
    z = dinv * (s_ref[0] + s_ref[1] + g_ref[...])
    h = jnp.maximum(
        jnp.dot(z, w_ref[...], preferred_element_type=jnp.float32) + b_ref[...],
        0.0,
    )
    o_ref[...] = dinv * h


def _tc_conv(s, g, deg2, W, b):
    return pl.pallas_call(
        _conv_body,
        grid=(N // RB,),
        in_specs=[
            pl.BlockSpec((NC, RB, D), lambda i: (0, i, 0)),
            pl.BlockSpec((RB, D), lambda i: (i, 0)),
            pl.BlockSpec((NC, RB, DEGW), lambda i: (0, i, 0)),
            pl.BlockSpec((D, D), lambda i: (0, 0)),
            pl.BlockSpec((1, D), lambda i: (0, 0)),
        ],
        out_specs=pl.BlockSpec((RB, D), lambda i: (i, 0)),
        out_shape=jax.ShapeDtypeStruct((N, D), jnp.float32),
    )(s, g, deg2, W, b)


def _final_body(s_ref, g_ref, d_ref, bat_ref, w3_ref, b3_ref, wd_ref, bd_ref, o_ref, acc_ref):
    i = pl.program_id(0)
    dinv = _dinv_block(d_ref)
    z = dinv * (s_ref[0] + s_ref[1] + g_ref[...])
    h3 = jnp.maximum(
        jnp.dot(z, w3_ref[...], preferred_element_type=jnp.float32) + b3_ref[...],
        0.0,
    )
    bat = bat_ref[0, 0, :]
    ind = (lax.broadcasted_iota(jnp.int32, (G, RB), 0) == bat[None, :]).astype(jnp.float32)
    part = jnp.dot(ind, h3, preferred_element_type=jnp.float32)

    @pl.when(i == 0)
    def _():
        acc_ref[...] = jnp.zeros_like(acc_ref)

    acc_ref[...] += part

    @pl.when(i == pl.num_programs(0) - 1)
    def _():
        o_ref[...] = (
            jnp.dot(acc_ref[...], wd_ref[...], preferred_element_type=jnp.float32)
            + bd_ref[...]
        )


def _tc_final(s, g, deg2, bat3, W3, b3, Wd, bd):
    return pl.pallas_call(
        _final_body,
        grid=(N // RB,),
        in_specs=[
            pl.BlockSpec((NC, RB, D), lambda i: (0, i, 0)),
            pl.BlockSpec((RB, D), lambda i: (i, 0)),
            pl.BlockSpec((NC, RB, DEGW), lambda i: (0, i, 0)),
            pl.BlockSpec((1, 1, RB), lambda i: (i, 0, 0)),
            pl.BlockSpec((D, D), lambda i: (0, 0)),
            pl.BlockSpec((1, D), lambda i: (0, 0)),
            pl.BlockSpec((D, D), lambda i: (0, 0)),
            pl.BlockSpec((1, D), lambda i: (0, 0)),
        ],
        out_specs=pl.BlockSpec((G, D), lambda i: (0, 0)),
        out_shape=jax.ShapeDtypeStruct((G, D), jnp.float32),
        scratch_shapes=[pltpu.VMEM((G, D), jnp.float32)],
    )(s, g, deg2, bat3, W3, b3, Wd, bd)


def kernel(x, edge_index, batch, W_enc, b_enc, W_c1, b_c1, W_c2, b_c2, W_c3, b_c3, W_dec, b_dec):
    src = edge_index[0]
    dst3 = edge_index[1].reshape(NT, NCH, CH)
    dst = edge_index[1].reshape(NT, NG, GCH, CH)
    deg2 = _sc_degree(dst3)
    g0 = _tc_encode(x, deg2, W_enc, b_enc.reshape(1, D))
    s = _sc_propagate(g0, src, dst)
    g1 = _tc_conv(s, g0, deg2, W_c1, b_c1.reshape(1, D))
    s = _sc_propagate(g1, src, dst)
    g2 = _tc_conv(s, g1, deg2, W_c2, b_c2.reshape(1, D))
    s = _sc_propagate(g2, src, dst)
    bat3 = batch.reshape(N // RB, 1, RB)
    return _tc_final(s, g2, deg2, bat3, W_c3, b_c3.reshape(1, D), W_dec, b_dec.reshape(1, D))
